# baseline (speedup 1.0000x reference)
import functools

import jax
import jax.numpy as jnp
from jax import lax
from jax.experimental import pallas as pl
from jax.experimental.pallas import tpu as pltpu
from jax.experimental.pallas import tpu_sc as plsc

N_TOTAL = 8 * 128 * 128 * 64
KSEL = float(int(N_TOTAL * 0.1))
NCORES = 2
NSUB = 16
NW = NCORES * NSUB
PER_W = N_TOTAL // NW
CHUNK = 8192
N_CHUNKS = PER_W // CHUNK
LANES = 16
NBINS = 32768
ROWS = 256
COLS = 128


NBUF = 3


N_SLABS = 16


def _hist_body(w_hbm, cnt_hbm, pbuf, hcnt, sp0, sp1, sp2):
    wid = lax.axis_index("s") * NCORES + lax.axis_index("c")
    base_slab = wid * N_SLABS
    psems = (sp0, sp1, sp2)

    def slab_ref(chunk):
        g = base_slab + chunk
        b_ = lax.shift_right_logical(g, 6)
        r_ = lax.bitwise_and(g, 63)
        return w_hbm.at[b_, r_]

    def issue(chunk, buf):
        pltpu.make_async_copy(slab_ref(chunk), pbuf.at[buf], psems[buf]).start()

    def wait(chunk, buf):
        pltpu.make_async_copy(slab_ref(chunk), pbuf.at[buf], psems[buf]).wait()

    for c0 in range(NBUF - 1):
        issue(c0, c0)

    def zero_body(i, carry):
        hcnt[pl.ds(i * LANES, LANES)] = jnp.zeros((LANES,), jnp.int32)
        return carry

    lax.fori_loop(0, NBINS // LANES, zero_body, 0)

    ones = jnp.ones((LANES,), jnp.int32)

    def compute(buf):
        def row_body(rr, carry):
            locs = [(2 * rr + (c // 4), (c % 4) * LANES) for c in range(8)]
            ws = [pbuf[buf, r, pl.ds(o, LANES)] for r, o in locs]
            bins = [lax.bitwise_and(w, 0x7FFF) for w in ws]
            bins += [lax.shift_right_logical(w, 16) for w in ws]
            for bin_ in bins:
                plsc.addupdate_scatter(hcnt, [bin_], ones)
            return carry

        lax.fori_loop(0, 64, row_body, 0)

    @pl.loop(0, N_SLABS, step=NBUF)
    def _chunk_loop(ci):
        for b in range(NBUF):
            chunk = ci + b

            @pl.when(chunk < N_SLABS)
            def _body():
                nxt = chunk + (NBUF - 1)

                @pl.when(nxt < N_SLABS)
                def _prefetch():
                    issue(nxt, (b + NBUF - 1) % NBUF)

                wait(chunk, b)
                compute(b)

    pltpu.sync_copy(hcnt, cnt_hbm.at[wid])


_sc_hist = functools.partial(
    pl.kernel,
    mesh=plsc.VectorSubcoreMesh(core_axis_name="c", subcore_axis_name="s"),
    out_type=jax.ShapeDtypeStruct((NW, NBINS), jnp.int32),
    scratch_types=[
        pltpu.VMEM((NBUF, 128, 64), jnp.int32),
        pltpu.VMEM((NBINS,), jnp.int32),
        pltpu.SemaphoreType.DMA,
        pltpu.SemaphoreType.DMA,
        pltpu.SemaphoreType.DMA,
    ],
    compiler_params=pltpu.CompilerParams(needs_layout_passes=False),
)(_hist_body)


def _select_body(cnt_ref, out_ref):
    c = jnp.sum(cnt_ref[...].astype(jnp.float32), axis=0)

    rr = lax.broadcasted_iota(jnp.int32, (ROWS, COLS), 0)
    cc = lax.broadcasted_iota(jnp.int32, (ROWS, COLS), 1)
    jglob = rr * COLS + cc
    v = lax.bitcast_convert_type((jglob << 16) | 0x8000, jnp.float32)
    s = c * v

    ii = lax.broadcasted_iota(jnp.int32, (COLS, COLS), 0)
    jj = lax.broadcasted_iota(jnp.int32, (COLS, COLS), 1)
    upper = (ii >= jj).astype(jnp.float32)
    ir_c = jnp.dot(c, upper, preferred_element_type=jnp.float32)
    ir_s = jnp.dot(s, upper, preferred_element_type=jnp.float32)

    rc = jnp.sum(c, axis=1, keepdims=True)
    rs = jnp.sum(s, axis=1, keepdims=True)
    i2 = lax.broadcasted_iota(jnp.int32, (ROWS, ROWS), 0)
    j2 = lax.broadcasted_iota(jnp.int32, (ROWS, ROWS), 1)
    strict = (j2 > i2).astype(jnp.float32)
    g_c = jnp.dot(strict, rc, preferred_element_type=jnp.float32)
    g_s = jnp.dot(strict, rs, preferred_element_type=jnp.float32)

    s_cnt = g_c + ir_c
    s_sum = g_s + ir_s

    mask = jnp.logical_and(s_cnt >= KSEL, (s_cnt - c) < KSEL)
    mf = mask.astype(jnp.float32)
    cb = jnp.sum(mf * c)
    sb = jnp.sum(mf * s)
    vb = jnp.sum(mf * v)
    scnt_b = jnp.sum(mf * s_cnt)
    ssum_b = jnp.sum(mf * s_sum)

    m = KSEL - (scnt_b - cb)
    loss = (ssum_b - sb + m * vb) / KSEL
    out_ref[...] = jnp.broadcast_to(loss, (1, 1))


def _pack_body(p_ref, t_ref, o_ref):
    d = jnp.abs(p_ref[...] - t_ref[...])
    b = lax.shift_right_logical(lax.bitcast_convert_type(d, jnp.int32), 16)
    o_ref[...] = b[:, 0:16] | (b[:, 16:32] << 16)


_pack = pl.pallas_call(
    _pack_body,
    grid=(8, 4),
    in_specs=[
        pl.BlockSpec((1, 32, 128, 64), lambda i, j: (i, j, 0, 0)),
        pl.BlockSpec((1, 32, 128, 64), lambda i, j: (i, j, 0, 0)),
    ],
    out_specs=pl.BlockSpec((1, 16, 128, 64), lambda i, j: (i, j, 0, 0)),
    out_shape=jax.ShapeDtypeStruct((8, 64, 128, 64), jnp.int32),
)


def kernel(preds, targets):
    cnt = _sc_hist(_pack(preds, targets))
    loss = pl.pallas_call(
        _select_body,
        out_shape=jax.ShapeDtypeStruct((1, 1), jnp.float32),
    )(cnt.reshape(NW, ROWS, COLS))
    return loss.reshape(())


# speedup vs baseline: 1.1245x; 1.1245x over previous
import functools

import jax
import jax.numpy as jnp
from jax import lax
from jax.experimental import pallas as pl
from jax.experimental.pallas import tpu as pltpu
from jax.experimental.pallas import tpu_sc as plsc

N_TOTAL = 8 * 128 * 128 * 64
KSEL = float(int(N_TOTAL * 0.1))
NCORES = 2
NSUB = 16
NW = NCORES * NSUB
PER_W = N_TOTAL // NW
CHUNK = 8192
N_CHUNKS = PER_W // CHUNK
LANES = 16
NBINS = 32768
ROWS = 256
COLS = 128


NBUF = 3


def _hist_body(p_hbm, t_hbm, cnt_hbm, pbuf, tbuf, hcnt,
               sp0, sp1, sp2, st0, st1, st2):
    wid = lax.axis_index("s") * NCORES + lax.axis_index("c")
    base_slab = wid * N_CHUNKS
    psems = (sp0, sp1, sp2)
    tsems = (st0, st1, st2)

    def slab_refs(chunk):
        g = base_slab + chunk
        b_ = lax.shift_right_logical(g, 7)
        r_ = lax.bitwise_and(g, 127)
        return p_hbm.at[b_, r_], t_hbm.at[b_, r_]

    def issue(chunk, buf):
        psrc, tsrc = slab_refs(chunk)
        pltpu.make_async_copy(psrc, pbuf.at[buf], psems[buf]).start()
        pltpu.make_async_copy(tsrc, tbuf.at[buf], tsems[buf]).start()

    def wait(chunk, buf):
        psrc, tsrc = slab_refs(chunk)
        pltpu.make_async_copy(psrc, pbuf.at[buf], psems[buf]).wait()
        pltpu.make_async_copy(tsrc, tbuf.at[buf], tsems[buf]).wait()

    for c0 in range(NBUF - 1):
        issue(c0, c0)

    def zero_body(i, carry):
        hcnt[pl.ds(i * LANES, LANES)] = jnp.zeros((LANES,), jnp.int32)
        return carry

    lax.fori_loop(0, NBINS // LANES, zero_body, 0)

    ones = jnp.ones((LANES,), jnp.int32)

    def compute(buf):
        def row_body(rr, carry):
            locs = [(2 * rr + (c // 4), (c % 4) * LANES) for c in range(8)]
            ps = [pbuf[buf, r, pl.ds(o, LANES)] for r, o in locs]
            ts = [tbuf[buf, r, pl.ds(o, LANES)] for r, o in locs]
            ds = [jnp.abs(pv - tv) for pv, tv in zip(ps, ts)]
            bins = [
                lax.shift_right_logical(lax.bitcast_convert_type(d, jnp.int32), 16)
                for d in ds
            ]
            for bin_ in bins:
                plsc.addupdate_scatter(hcnt, [bin_], ones)
            return carry

        lax.fori_loop(0, 64, row_body, 0)

    @pl.loop(0, N_CHUNKS, step=NBUF)
    def _chunk_loop(ci):
        for b in range(NBUF):
            chunk = ci + b

            @pl.when(chunk < N_CHUNKS)
            def _body():
                nxt = chunk + (NBUF - 1)

                @pl.when(nxt < N_CHUNKS)
                def _prefetch():
                    issue(nxt, (b + NBUF - 1) % NBUF)

                wait(chunk, b)
                compute(b)

    pltpu.sync_copy(hcnt, cnt_hbm.at[wid])


_sc_hist = functools.partial(
    pl.kernel,
    mesh=plsc.VectorSubcoreMesh(core_axis_name="c", subcore_axis_name="s"),
    out_type=jax.ShapeDtypeStruct((NW, NBINS), jnp.int32),
    scratch_types=[
        pltpu.VMEM((NBUF, 128, 64), jnp.float32),
        pltpu.VMEM((NBUF, 128, 64), jnp.float32),
        pltpu.VMEM((NBINS,), jnp.int32),
        pltpu.SemaphoreType.DMA,
        pltpu.SemaphoreType.DMA,
        pltpu.SemaphoreType.DMA,
        pltpu.SemaphoreType.DMA,
        pltpu.SemaphoreType.DMA,
        pltpu.SemaphoreType.DMA,
    ],
    compiler_params=pltpu.CompilerParams(needs_layout_passes=False),
)(_hist_body)


def _select_body(cnt_ref, out_ref):
    c = jnp.sum(cnt_ref[...].astype(jnp.float32), axis=0)

    rr = lax.broadcasted_iota(jnp.int32, (ROWS, COLS), 0)
    cc = lax.broadcasted_iota(jnp.int32, (ROWS, COLS), 1)
    jglob = rr * COLS + cc
    v = lax.bitcast_convert_type((jglob << 16) | 0x8000, jnp.float32)
    s = c * v

    ii = lax.broadcasted_iota(jnp.int32, (COLS, COLS), 0)
    jj = lax.broadcasted_iota(jnp.int32, (COLS, COLS), 1)
    upper = (ii >= jj).astype(jnp.float32)
    ir_c = jnp.dot(c, upper, preferred_element_type=jnp.float32)
    ir_s = jnp.dot(s, upper, preferred_element_type=jnp.float32)

    rc = jnp.sum(c, axis=1, keepdims=True)
    rs = jnp.sum(s, axis=1, keepdims=True)
    i2 = lax.broadcasted_iota(jnp.int32, (ROWS, ROWS), 0)
    j2 = lax.broadcasted_iota(jnp.int32, (ROWS, ROWS), 1)
    strict = (j2 > i2).astype(jnp.float32)
    g_c = jnp.dot(strict, rc, preferred_element_type=jnp.float32)
    g_s = jnp.dot(strict, rs, preferred_element_type=jnp.float32)

    s_cnt = g_c + ir_c
    s_sum = g_s + ir_s

    mask = jnp.logical_and(s_cnt >= KSEL, (s_cnt - c) < KSEL)
    mf = mask.astype(jnp.float32)
    cb = jnp.sum(mf * c)
    sb = jnp.sum(mf * s)
    vb = jnp.sum(mf * v)
    scnt_b = jnp.sum(mf * s_cnt)
    ssum_b = jnp.sum(mf * s_sum)

    m = KSEL - (scnt_b - cb)
    loss = (ssum_b - sb + m * vb) / KSEL
    out_ref[...] = jnp.broadcast_to(loss, (1, 1))


def kernel(preds, targets):
    cnt = _sc_hist(preds, targets)
    loss = pl.pallas_call(
        _select_body,
        out_shape=jax.ShapeDtypeStruct((1, 1), jnp.float32),
    )(cnt.reshape(NW, ROWS, COLS))
    return loss.reshape(())
